# Initial kernel scaffold; baseline (speedup 1.0000x reference)
import jax
import jax.numpy as jnp
from jax.experimental import pallas as pl


def kernel(x, edge_index_rel0, edge_index_rel1, W0_r0, b0_r0, W0_r1, b0_r1, W1_r0, b1_r0, W1_r1, b1_r1, W2_r0, b2_r0, W2_r1, b2_r1):
    raise NotImplementedError("write your pallas kernel here")


# speedup vs baseline: 4.1560x; 4.1560x over previous
import functools

import jax
import jax.numpy as jnp
from jax import lax
from jax.experimental import pallas as pl
from jax.experimental.pallas import tpu as pltpu
from jax.experimental.pallas import tpu_sc as plsc

_N = 10000
_E = 160000
_D = 256
_DH = 128
_NC = 2
_NS = 16
_CH = 128
_NCHUNK = _E // _CH
_STRIPE = 640
_LAST = _N - (_NS - 1) * _STRIPE
_ROWB = 1000
_NB = _N // _ROWB
_NP = 10112
_DSTRIPE = 640
_DLAST = _NP - (_NS - 1) * _DSTRIPE

_f32 = jnp.float32


@functools.lru_cache(maxsize=None)
def _sc_mesh():
    return plsc.VectorSubcoreMesh(
        core_axis_name="c", subcore_axis_name="s",
        num_cores=_NC, num_subcores=_NS)


def _copy_stripes(src_sh, dst_hbm, s):
    @pl.when(s < _NS - 1)
    def _():
        pltpu.sync_copy(src_sh.at[pl.ds(s * _STRIPE, _STRIPE)],
                        dst_hbm.at[pl.ds(s * _STRIPE, _STRIPE)])

    @pl.when(s == _NS - 1)
    def _():
        pltpu.sync_copy(src_sh.at[pl.ds((_NS - 1) * _STRIPE, _LAST)],
                        dst_hbm.at[pl.ds((_NS - 1) * _STRIPE, _LAST)])


def _zero_acc_stripes(z_hbm, acc_sh, s):
    @pl.when(s < _NS - 1)
    def _():
        pltpu.sync_copy(z_hbm, acc_sh.at[pl.ds(s * _STRIPE, _STRIPE)])

    @pl.when(s == _NS - 1)
    def _():
        pltpu.sync_copy(z_hbm.at[pl.ds(0, _LAST)],
                        acc_sh.at[pl.ds((_NS - 1) * _STRIPE, _LAST)])


def _deg_body(s0, d0, s1, d1, zeros_n, ones_c, out,
              idx_v, ones_v, sh0, sh1, sh2, sh3):
    c = lax.axis_index("c")
    s = lax.axis_index("s")
    wid = c * _NS + s

    @pl.when(s == 0)
    def _():
        pltpu.sync_copy(zeros_n, sh0)
        pltpu.sync_copy(zeros_n, sh1)
        pltpu.sync_copy(zeros_n, sh2)
        pltpu.sync_copy(zeros_n, sh3)

    pltpu.sync_copy(ones_c, ones_v)
    plsc.subcore_barrier()

    for ei, sh in ((s0, sh0), (d0, sh1), (s1, sh2), (d1, sh3)):
        def body(k, _, ei=ei, sh=sh):
            cid = wid + k * (_NC * _NS)

            @pl.when(cid < _NCHUNK)
            def _():
                pltpu.sync_copy(ei.at[pl.ds(cid * _CH, _CH)], idx_v)
                pltpu.sync_copy(ones_v, sh.at[idx_v], add=True)

            return 0

        lax.fori_loop(0, (_NCHUNK + _NC * _NS - 1) // (_NC * _NS), body, 0)

    plsc.subcore_barrier()
    for a, sh in enumerate((sh0, sh1, sh2, sh3)):
        @pl.when(s < _NS - 1)
        def _(sh=sh, a=a):
            pltpu.sync_copy(sh.at[pl.ds(s * _DSTRIPE, _DSTRIPE)],
                            out.at[c, a, pl.ds(s * _DSTRIPE, _DSTRIPE)])

        @pl.when(s == _NS - 1)
        def _(sh=sh, a=a):
            pltpu.sync_copy(sh.at[pl.ds((_NS - 1) * _DSTRIPE, _DLAST)],
                            out.at[c, a, pl.ds((_NS - 1) * _DSTRIPE, _DLAST)])


@functools.lru_cache(maxsize=None)
def _deg_call():
    return pl.kernel(
        _deg_body,
        out_type=jax.ShapeDtypeStruct((_NC, 4, _NP), _f32),
        mesh=_sc_mesh(),
        scratch_types=[
            pltpu.VMEM((_CH,), jnp.int32),
            pltpu.VMEM((_CH,), _f32),
            pltpu.VMEM_SHARED((_NP,), _f32),
            pltpu.VMEM_SHARED((_NP,), _f32),
            pltpu.VMEM_SHARED((_NP,), _f32),
            pltpu.VMEM_SHARED((_NP,), _f32),
        ],
    )


def _scale_body(degp_ref, so_ref, si_ref):
    deg = degp_ref[0] + degp_ref[1]
    sc = lax.rsqrt(jnp.maximum(deg, 1.0))
    so_ref[...] = jnp.stack([sc[0, :_N], sc[2, :_N]], axis=1)
    si_ref[...] = jnp.stack([sc[1, :_N], sc[3, :_N]], axis=1)


_scale_call = pl.pallas_call(
    _scale_body,
    out_shape=(jax.ShapeDtypeStruct((_N, 2), _f32),
               jax.ShapeDtypeStruct((_N, 2), _f32)),
)


def _mm0_body(x_ref, so_ref, w0_ref, w1_ref, p0a, p0b, p1a, p1b):
    xb = x_ref[...]
    for r, (w_ref, pa, pb) in enumerate(((w0_ref, p0a, p0b),
                                         (w1_ref, p1a, p1b))):
        h = xb * so_ref[:, r:r + 1]
        p = jnp.dot(h, w_ref[...], preferred_element_type=_f32)
        pa[...] = p[:, :_DH]
        pb[...] = p[:, _DH:]


_mm0_call = pl.pallas_call(
    _mm0_body,
    grid=(_NB,),
    in_specs=[
        pl.BlockSpec((_ROWB, _D), lambda j: (j, 0)),
        pl.BlockSpec((_ROWB, 2), lambda j: (j, 0)),
        pl.BlockSpec((_D, _D), lambda j: (0, 0)),
        pl.BlockSpec((_D, _D), lambda j: (0, 0)),
    ],
    out_specs=[pl.BlockSpec((_ROWB, _DH), lambda j: (j, 0))] * 4,
    out_shape=[jax.ShapeDtypeStruct((_N, _DH), _f32)] * 4,
)


def _mm_mid_body(split, a0a, a0b, a1a, a1b, si_ref, b_ref, so_ref,
                 w0_ref, w1_ref, *outs):
    si0 = si_ref[:, 0:1]
    si1 = si_ref[:, 1:2]
    ha = si0 * a0a[...] + si1 * a1a[...] + b_ref[0, :_DH]
    hb = si0 * a0b[...] + si1 * a1b[...] + b_ref[0, _DH:]
    h = jnp.concatenate([ha, hb], axis=1)
    for r, w_ref in enumerate((w0_ref, w1_ref)):
        p = jnp.dot(h * so_ref[:, r:r + 1], w_ref[...],
                    preferred_element_type=_f32)
        if split:
            outs[2 * r][...] = p[:, :_DH]
            outs[2 * r + 1][...] = p[:, _DH:]
        else:
            outs[r][...] = p


def _make_mm_mid(split, dout):
    n_out = 4 if split else 2
    ob = dout // 2 if split else dout
    return pl.pallas_call(
        functools.partial(_mm_mid_body, split),
        grid=(_NB,),
        in_specs=[
            pl.BlockSpec((_ROWB, _DH), lambda j: (j, 0)),
            pl.BlockSpec((_ROWB, _DH), lambda j: (j, 0)),
            pl.BlockSpec((_ROWB, _DH), lambda j: (j, 0)),
            pl.BlockSpec((_ROWB, _DH), lambda j: (j, 0)),
            pl.BlockSpec((_ROWB, 2), lambda j: (j, 0)),
            pl.BlockSpec((1, _D), lambda j: (0, 0)),
            pl.BlockSpec((_ROWB, 2), lambda j: (j, 0)),
            pl.BlockSpec((_D, dout), lambda j: (0, 0)),
            pl.BlockSpec((_D, dout), lambda j: (0, 0)),
        ],
        out_specs=[pl.BlockSpec((_ROWB, ob), lambda j: (j, 0))] * n_out,
        out_shape=[jax.ShapeDtypeStruct((_N, ob), _f32)] * n_out,
    )


_mm_mid_call = _make_mm_mid(True, _D)
_mm_last_call = _make_mm_mid(False, _DH)


def _fin_body(p0A, p0B, p1A, p1B, si_ref, b_ref, out_ref):
    si0 = si_ref[:, 0:1]
    si1 = si_ref[:, 1:2]
    out_ref[...] = (si0 * (p0A[...] + p0B[...])
                    + si1 * (p1A[...] + p1B[...]) + b_ref[0, :])


_fin_call = pl.pallas_call(
    _fin_body,
    grid=(_NB,),
    in_specs=[
        pl.BlockSpec((_ROWB, _DH), lambda j: (j, 0)),
        pl.BlockSpec((_ROWB, _DH), lambda j: (j, 0)),
        pl.BlockSpec((_ROWB, _DH), lambda j: (j, 0)),
        pl.BlockSpec((_ROWB, _DH), lambda j: (j, 0)),
        pl.BlockSpec((_ROWB, 2), lambda j: (j, 0)),
        pl.BlockSpec((1, _DH), lambda j: (0, 0)),
    ],
    out_specs=pl.BlockSpec((_ROWB, _DH), lambda j: (j, 0)),
    out_shape=jax.ShapeDtypeStruct((_N, _DH), _f32),
)


def _prop256_body(ta, tb, src, dst, z2, out_a, out_b,
                  idx_s, idx_d, rows, acc, sem):
    c = lax.axis_index("c")
    s = lax.axis_index("s")

    _zero_acc_stripes(z2, acc, s)
    plsc.subcore_barrier()

    def chunk(k, _):
        cid = s + k * _NS

        @pl.when(cid < _NCHUNK)
        def _():
            pltpu.sync_copy(src.at[pl.ds(cid * _CH, _CH)], idx_s)
            pltpu.sync_copy(dst.at[pl.ds(cid * _CH, _CH)], idx_d)

            @pl.when(c == 0)
            def _():
                pltpu.async_copy(ta.at[idx_s], rows, sem).wait()

            @pl.when(c == 1)
            def _():
                pltpu.async_copy(tb.at[idx_s], rows, sem).wait()

            pltpu.sync_copy(rows, acc.at[idx_d], add=True)

        return 0

    lax.fori_loop(0, (_NCHUNK + _NS - 1) // _NS, chunk, 0)
    plsc.subcore_barrier()

    @pl.when(c == 0)
    def _():
        _copy_stripes(acc, out_a, s)

    @pl.when(c == 1)
    def _():
        _copy_stripes(acc, out_b, s)


@functools.lru_cache(maxsize=None)
def _prop256_call():
    return pl.kernel(
        _prop256_body,
        out_type=(jax.ShapeDtypeStruct((_N, _DH), _f32),
                  jax.ShapeDtypeStruct((_N, _DH), _f32)),
        mesh=_sc_mesh(),
        scratch_types=[
            pltpu.VMEM((_CH,), jnp.int32),
            pltpu.VMEM((_CH,), jnp.int32),
            pltpu.VMEM((_CH, _DH), _f32),
            pltpu.VMEM_SHARED((_N, _DH), _f32),
            pltpu.SemaphoreType.DMA,
        ],
    )


def _prop128_body(tab, src, dst, z2, out_a, out_b,
                  idx_s, idx_d, rows, acc, sem):
    c = lax.axis_index("c")
    s = lax.axis_index("s")
    wid = c * _NS + s

    _zero_acc_stripes(z2, acc, s)
    plsc.subcore_barrier()

    def chunk(k, _):
        cid = wid + k * (_NC * _NS)

        @pl.when(cid < _NCHUNK)
        def _():
            pltpu.sync_copy(src.at[pl.ds(cid * _CH, _CH)], idx_s)
            pltpu.sync_copy(dst.at[pl.ds(cid * _CH, _CH)], idx_d)
            pltpu.async_copy(tab.at[idx_s], rows, sem).wait()
            pltpu.sync_copy(rows, acc.at[idx_d], add=True)

        return 0

    lax.fori_loop(0, (_NCHUNK + _NC * _NS - 1) // (_NC * _NS), chunk, 0)
    plsc.subcore_barrier()

    @pl.when(c == 0)
    def _():
        _copy_stripes(acc, out_a, s)

    @pl.when(c == 1)
    def _():
        _copy_stripes(acc, out_b, s)


@functools.lru_cache(maxsize=None)
def _prop128_call():
    return pl.kernel(
        _prop128_body,
        out_type=(jax.ShapeDtypeStruct((_N, _DH), _f32),
                  jax.ShapeDtypeStruct((_N, _DH), _f32)),
        mesh=_sc_mesh(),
        scratch_types=[
            pltpu.VMEM((_CH,), jnp.int32),
            pltpu.VMEM((_CH,), jnp.int32),
            pltpu.VMEM((_CH, _DH), _f32),
            pltpu.VMEM_SHARED((_N, _DH), _f32),
            pltpu.SemaphoreType.DMA,
        ],
    )


def kernel(x, edge_index_rel0, edge_index_rel1,
           W0_r0, b0_r0, W0_r1, b0_r1,
           W1_r0, b1_r0, W1_r1, b1_r1,
           W2_r0, b2_r0, W2_r1, b2_r1):
    s0 = edge_index_rel0[0]
    d0 = edge_index_rel0[1]
    s1 = edge_index_rel1[0]
    d1 = edge_index_rel1[1]

    zeros_n = jnp.zeros((_NP,), _f32)
    ones_c = jnp.ones((_CH,), _f32)
    z2 = jnp.zeros((_STRIPE, _DH), _f32)

    degp = _deg_call()(s0, d0, s1, d1, zeros_n, ones_c)
    so, si = _scale_call(degp)

    b0 = (b0_r0 + b0_r1).reshape(1, _D)
    b1 = (b1_r0 + b1_r1).reshape(1, _D)
    b2 = (b2_r0 + b2_r1).reshape(1, _DH)

    prop256 = _prop256_call()
    prop128 = _prop128_call()

    p0a, p0b, p1a, p1b = _mm0_call(x, so, W0_r0, W0_r1)
    a0a, a0b = prop256(p0a, p0b, s0, d0, z2)
    a1a, a1b = prop256(p1a, p1b, s1, d1, z2)

    q0a, q0b, q1a, q1b = _mm_mid_call(a0a, a0b, a1a, a1b, si, b0, so,
                                      W1_r0, W1_r1)
    a0a, a0b = prop256(q0a, q0b, s0, d0, z2)
    a1a, a1b = prop256(q1a, q1b, s1, d1, z2)

    p0, p1 = _mm_last_call(a0a, a0b, a1a, a1b, si, b1, so, W2_r0, W2_r1)
    r0A, r0B = prop128(p0, s0, d0, z2)
    r1A, r1B = prop128(p1, s1, d1, z2)

    return _fin_call(r0A, r0B, r1A, r1B, si, b2)


# speedup vs baseline: 6.8160x; 1.6401x over previous
import functools

import jax
import jax.numpy as jnp
from jax import lax
from jax.experimental import pallas as pl
from jax.experimental.pallas import tpu as pltpu
from jax.experimental.pallas import tpu_sc as plsc

_N = 10000
_E = 160000
_D = 256
_DH = 128
_NC = 2
_NS = 16
_CH = 128
_NCHP = 1280
_EP = _NCHP * _CH
_NROWS = 10016
_NBUF = 2
_PCH = 40
_STRIPE = 640
_LAST = _N - (_NS - 1) * _STRIPE
_ROWB = 1000
_NB = _N // _ROWB
_NP = 10112
_DSTRIPE = 640
_DLAST = _NP - (_NS - 1) * _DSTRIPE

_f32 = jnp.float32


@functools.lru_cache(maxsize=None)
def _sc_mesh():
    return plsc.VectorSubcoreMesh(
        core_axis_name="c", subcore_axis_name="s",
        num_cores=_NC, num_subcores=_NS)


def _copy_stripes(src_sh, dst_hbm, s):
    @pl.when(s < _NS - 1)
    def _():
        pltpu.sync_copy(src_sh.at[pl.ds(s * _STRIPE, _STRIPE)],
                        dst_hbm.at[pl.ds(s * _STRIPE, _STRIPE)])

    @pl.when(s == _NS - 1)
    def _():
        pltpu.sync_copy(src_sh.at[pl.ds((_NS - 1) * _STRIPE, _LAST)],
                        dst_hbm.at[pl.ds((_NS - 1) * _STRIPE, _LAST)])


def _zero_acc_stripes(z_hbm, acc_sh, s):
    @pl.when(s < _NS - 1)
    def _():
        pltpu.sync_copy(z_hbm, acc_sh.at[pl.ds(s * _STRIPE, _STRIPE)])

    @pl.when(s == _NS - 1)
    def _():
        pltpu.sync_copy(z_hbm.at[pl.ds(0, _LAST)],
                        acc_sh.at[pl.ds((_NS - 1) * _STRIPE, _LAST)])


_DCPW = _NCHP // (_NC * _NS)


def _deg_body(s0, d0, s1, d1, zeros_n, ones_c, out,
              idx_v, ones_v, sh0, sh1, sh2, sh3, sem):
    c = lax.axis_index("c")
    s = lax.axis_index("s")
    wid = c * _NS + s

    @pl.when(s == 0)
    def _():
        pltpu.sync_copy(zeros_n, sh0)
        pltpu.sync_copy(zeros_n, sh1)
        pltpu.sync_copy(zeros_n, sh2)
        pltpu.sync_copy(zeros_n, sh3)

    pltpu.sync_copy(ones_c, ones_v)
    plsc.subcore_barrier()

    for ei, sh in ((s0, sh0), (d0, sh1), (s1, sh2), (d1, sh3)):
        pltpu.sync_copy(ei.at[pl.ds(wid * _DCPW, _DCPW)], idx_v)

        def grp(g, _, sh=sh):
            descs = []
            for b in range(10):
                descs.append(pltpu.async_copy(
                    ones_v, sh.at[idx_v.at[g * 10 + b]], sem, add=True))
            for d in descs:
                d.wait()
            return 0

        lax.fori_loop(0, _DCPW // 10, grp, 0)

    plsc.subcore_barrier()
    for a, sh in enumerate((sh0, sh1, sh2, sh3)):
        @pl.when(s < _NS - 1)
        def _(sh=sh, a=a):
            pltpu.sync_copy(sh.at[pl.ds(s * _DSTRIPE, _DSTRIPE)],
                            out.at[c, a, pl.ds(s * _DSTRIPE, _DSTRIPE)])

        @pl.when(s == _NS - 1)
        def _(sh=sh, a=a):
            pltpu.sync_copy(sh.at[pl.ds((_NS - 1) * _DSTRIPE, _DLAST)],
                            out.at[c, a, pl.ds((_NS - 1) * _DSTRIPE, _DLAST)])


@functools.lru_cache(maxsize=None)
def _deg_call():
    return pl.kernel(
        _deg_body,
        out_type=jax.ShapeDtypeStruct((_NC, 4, _NP), _f32),
        mesh=_sc_mesh(),
        scratch_types=[
            pltpu.VMEM((_DCPW, _CH), jnp.int32),
            pltpu.VMEM((_CH,), _f32),
            pltpu.VMEM_SHARED((_NP,), _f32),
            pltpu.VMEM_SHARED((_NP,), _f32),
            pltpu.VMEM_SHARED((_NP,), _f32),
            pltpu.VMEM_SHARED((_NP,), _f32),
            pltpu.SemaphoreType.DMA,
        ],
    )


def _scale_body(degp_ref, so_ref, si_ref):
    deg = degp_ref[0] + degp_ref[1]
    sc = lax.rsqrt(jnp.maximum(deg, 1.0))
    so_ref[...] = jnp.stack([sc[0, :_N], sc[2, :_N]], axis=1)
    si_ref[...] = jnp.stack([sc[1, :_N], sc[3, :_N]], axis=1)


_scale_call = pl.pallas_call(
    _scale_body,
    out_shape=(jax.ShapeDtypeStruct((_N, 2), _f32),
               jax.ShapeDtypeStruct((_N, 2), _f32)),
)


def _mm0_body(x_ref, so_ref, w0_ref, w1_ref, p0a, p0b, p1a, p1b):
    xb = x_ref[...]
    for r, (w_ref, pa, pb) in enumerate(((w0_ref, p0a, p0b),
                                         (w1_ref, p1a, p1b))):
        h = xb * so_ref[:, r:r + 1]
        p = jnp.dot(h, w_ref[...], preferred_element_type=_f32)
        pa[...] = p[:, :_DH]
        pb[...] = p[:, _DH:]


_mm0_call = pl.pallas_call(
    _mm0_body,
    grid=(_NB,),
    in_specs=[
        pl.BlockSpec((_ROWB, _D), lambda j: (j, 0)),
        pl.BlockSpec((_ROWB, 2), lambda j: (j, 0)),
        pl.BlockSpec((_D, _D), lambda j: (0, 0)),
        pl.BlockSpec((_D, _D), lambda j: (0, 0)),
    ],
    out_specs=[pl.BlockSpec((_ROWB, _DH), lambda j: (j, 0))] * 4,
    out_shape=[jax.ShapeDtypeStruct((_NROWS, _DH), _f32)] * 4,
)


def _mm_mid_body(split, a0a, a0b, a1a, a1b, si_ref, b_ref, so_ref,
                 w0_ref, w1_ref, *outs):
    si0 = si_ref[:, 0:1]
    si1 = si_ref[:, 1:2]
    ha = si0 * a0a[...] + si1 * a1a[...] + b_ref[0, :_DH]
    hb = si0 * a0b[...] + si1 * a1b[...] + b_ref[0, _DH:]
    h = jnp.concatenate([ha, hb], axis=1)
    for r, w_ref in enumerate((w0_ref, w1_ref)):
        p = jnp.dot(h * so_ref[:, r:r + 1], w_ref[...],
                    preferred_element_type=_f32)
        if split:
            outs[2 * r][...] = p[:, :_DH]
            outs[2 * r + 1][...] = p[:, _DH:]
        else:
            outs[r][...] = p


def _make_mm_mid(split, dout):
    n_out = 4 if split else 2
    ob = dout // 2 if split else dout
    return pl.pallas_call(
        functools.partial(_mm_mid_body, split),
        grid=(_NB,),
        in_specs=[
            pl.BlockSpec((_ROWB, _DH), lambda j: (j, 0)),
            pl.BlockSpec((_ROWB, _DH), lambda j: (j, 0)),
            pl.BlockSpec((_ROWB, _DH), lambda j: (j, 0)),
            pl.BlockSpec((_ROWB, _DH), lambda j: (j, 0)),
            pl.BlockSpec((_ROWB, 2), lambda j: (j, 0)),
            pl.BlockSpec((1, _D), lambda j: (0, 0)),
            pl.BlockSpec((_ROWB, 2), lambda j: (j, 0)),
            pl.BlockSpec((_D, dout), lambda j: (0, 0)),
            pl.BlockSpec((_D, dout), lambda j: (0, 0)),
        ],
        out_specs=[pl.BlockSpec((_ROWB, ob), lambda j: (j, 0))] * n_out,
        out_shape=[jax.ShapeDtypeStruct((_NROWS, ob), _f32)] * n_out,
    )


_mm_mid_call = _make_mm_mid(True, _D)
_mm_last_call = _make_mm_mid(False, _DH)


def _fin_body(p0A, p0B, p1A, p1B, si_ref, b_ref, out_ref):
    si0 = si_ref[:, 0:1]
    si1 = si_ref[:, 1:2]
    out_ref[...] = (si0 * (p0A[...] + p0B[...])
                    + si1 * (p1A[...] + p1B[...]) + b_ref[0, :])


_fin_call = pl.pallas_call(
    _fin_body,
    grid=(_NB,),
    in_specs=[
        pl.BlockSpec((_ROWB, _DH), lambda j: (j, 0)),
        pl.BlockSpec((_ROWB, _DH), lambda j: (j, 0)),
        pl.BlockSpec((_ROWB, _DH), lambda j: (j, 0)),
        pl.BlockSpec((_ROWB, _DH), lambda j: (j, 0)),
        pl.BlockSpec((_ROWB, 2), lambda j: (j, 0)),
        pl.BlockSpec((1, _DH), lambda j: (0, 0)),
    ],
    out_specs=pl.BlockSpec((_ROWB, _DH), lambda j: (j, 0)),
    out_shape=jax.ShapeDtypeStruct((_N, _DH), _f32),
)


def _prop_pipeline(tab, acc, src_v, dst_v, cpw, bufs, gsems, ssems):
    nsteps = cpw // _NBUF

    for b in range(_NBUF):
        pltpu.async_copy(tab.at[src_v.at[b]], bufs[b], gsems[b])

    def step(k, _):
        for b in range(_NBUF):
            c = k * _NBUF + b
            pltpu.make_async_copy(tab.at[src_v.at[c]], bufs[b],
                                  gsems[b]).wait()
            pltpu.async_copy(bufs[b], acc.at[dst_v.at[c]], ssems[b],
                             add=True)
        for b in range(_NBUF):
            c = k * _NBUF + b
            pltpu.make_async_copy(bufs[b], acc.at[dst_v.at[c]],
                                  ssems[b]).wait()

            @pl.when(k < nsteps - 1)
            def _(b=b, c=c):
                pltpu.async_copy(tab.at[src_v.at[c + _NBUF]], bufs[b],
                                 gsems[b])

        return 0

    lax.fori_loop(0, nsteps, step, 0)


_CPW256 = _NCHP // _NS


def _prop256_body(ta, tb, src, dst, z2, out_a, out_b,
                  idx_s, idx_d, b0, b1,
                  acc, g0, g1, s0, s1):
    c = lax.axis_index("c")
    s = lax.axis_index("s")

    _zero_acc_stripes(z2, acc, s)
    plsc.subcore_barrier()

    bufs = (b0, b1)
    gsems = (g0, g1)
    ssems = (s0, s1)

    for phase in range(_CPW256 // _PCH):
        base = s * _CPW256 + phase * _PCH
        pltpu.sync_copy(src.at[pl.ds(base, _PCH)], idx_s)
        pltpu.sync_copy(dst.at[pl.ds(base, _PCH)], idx_d)

        @pl.when(c == 0)
        def _():
            _prop_pipeline(ta, acc, idx_s, idx_d, _PCH, bufs, gsems, ssems)

        @pl.when(c == 1)
        def _():
            _prop_pipeline(tb, acc, idx_s, idx_d, _PCH, bufs, gsems, ssems)

    plsc.subcore_barrier()

    @pl.when(c == 0)
    def _():
        _copy_stripes(acc, out_a, s)

    @pl.when(c == 1)
    def _():
        _copy_stripes(acc, out_b, s)


def _prop_scratch():
    return ([
        pltpu.VMEM((_PCH, _CH), jnp.int32),
        pltpu.VMEM((_PCH, _CH), jnp.int32),
    ] + [pltpu.VMEM((_CH, _DH), _f32)] * _NBUF
      + [pltpu.VMEM_SHARED((_NROWS, _DH), _f32)]
      + [pltpu.SemaphoreType.DMA] * (2 * _NBUF))


@functools.lru_cache(maxsize=None)
def _prop256_call():
    return pl.kernel(
        _prop256_body,
        out_type=(jax.ShapeDtypeStruct((_N, _DH), _f32),
                  jax.ShapeDtypeStruct((_N, _DH), _f32)),
        mesh=_sc_mesh(),
        scratch_types=_prop_scratch(),
    )


_CPW128 = _NCHP // (_NC * _NS)


def _prop128_body(tab, src, dst, z2, out_a, out_b,
                  idx_s, idx_d, b0, b1,
                  acc, g0, g1, s0, s1):
    c = lax.axis_index("c")
    s = lax.axis_index("s")
    wid = c * _NS + s

    _zero_acc_stripes(z2, acc, s)
    pltpu.sync_copy(src.at[pl.ds(wid * _CPW128, _CPW128)], idx_s)
    pltpu.sync_copy(dst.at[pl.ds(wid * _CPW128, _CPW128)], idx_d)
    plsc.subcore_barrier()

    _prop_pipeline(tab, acc, idx_s, idx_d, _CPW128,
                   (b0, b1), (g0, g1), (s0, s1))

    plsc.subcore_barrier()

    @pl.when(c == 0)
    def _():
        _copy_stripes(acc, out_a, s)

    @pl.when(c == 1)
    def _():
        _copy_stripes(acc, out_b, s)


@functools.lru_cache(maxsize=None)
def _prop128_call():
    return pl.kernel(
        _prop128_body,
        out_type=(jax.ShapeDtypeStruct((_N, _DH), _f32),
                  jax.ShapeDtypeStruct((_N, _DH), _f32)),
        mesh=_sc_mesh(),
        scratch_types=_prop_scratch(),
    )


def kernel(x, edge_index_rel0, edge_index_rel1,
           W0_r0, b0_r0, W0_r1, b0_r1,
           W1_r0, b1_r0, W1_r1, b1_r1,
           W2_r0, b2_r0, W2_r1, b2_r1):
    pad = _N + (jnp.arange(_EP - _E, dtype=jnp.int32) % (_NROWS - _N))

    def _padr(a):
        return jnp.concatenate([a, pad]).reshape(_NCHP, _CH)

    s0 = _padr(edge_index_rel0[0])
    d0 = _padr(edge_index_rel0[1])
    s1 = _padr(edge_index_rel1[0])
    d1 = _padr(edge_index_rel1[1])

    zeros_n = jnp.zeros((_NP,), _f32)
    ones_c = jnp.ones((_CH,), _f32)
    z2 = jnp.zeros((_STRIPE, _DH), _f32)

    degp = _deg_call()(s0, d0, s1, d1, zeros_n, ones_c)
    so, si = _scale_call(degp)

    b0 = (b0_r0 + b0_r1).reshape(1, _D)
    b1 = (b1_r0 + b1_r1).reshape(1, _D)
    b2 = (b2_r0 + b2_r1).reshape(1, _DH)

    prop256 = _prop256_call()
    prop128 = _prop128_call()

    p0a, p0b, p1a, p1b = _mm0_call(x, so, W0_r0, W0_r1)
    a0a, a0b = prop256(p0a, p0b, s0, d0, z2)
    a1a, a1b = prop256(p1a, p1b, s1, d1, z2)

    q0a, q0b, q1a, q1b = _mm_mid_call(a0a, a0b, a1a, a1b, si, b0, so,
                                      W1_r0, W1_r1)
    a0a, a0b = prop256(q0a, q0b, s0, d0, z2)
    a1a, a1b = prop256(q1a, q1b, s1, d1, z2)

    p0, p1 = _mm_last_call(a0a, a0b, a1a, a1b, si, b1, so, W2_r0, W2_r1)
    r0A, r0B = prop128(p0, s0, d0, z2)
    r1A, r1B = prop128(p1, s1, d1, z2)

    return _fin_call(r0A, r0B, r1A, r1B, si, b2)


# speedup vs baseline: 8.5449x; 1.2537x over previous
import functools

import jax
import jax.numpy as jnp
from jax import lax
from jax.experimental import pallas as pl
from jax.experimental.pallas import tpu as pltpu
from jax.experimental.pallas import tpu_sc as plsc

_N = 10000
_E = 160000
_D = 256
_DH = 128
_NC = 2
_NS = 16
_CH = 128
_NCHP = 1280
_EP = _NCHP * _CH
_NROWS = 10016
_NBUF = 2
_PCH = 40
_STRIPE = 640
_LAST = _N - (_NS - 1) * _STRIPE
_ROWB = 1000
_NB = _N // _ROWB
_NP = 10112
_DSTRIPE = 640
_DLAST = _NP - (_NS - 1) * _DSTRIPE

_f32 = jnp.float32


@functools.lru_cache(maxsize=None)
def _sc_mesh():
    return plsc.VectorSubcoreMesh(
        core_axis_name="c", subcore_axis_name="s",
        num_cores=_NC, num_subcores=_NS)


def _copy_stripes(src_sh, dst_hbm, s):
    @pl.when(s < _NS - 1)
    def _():
        pltpu.sync_copy(src_sh.at[pl.ds(s * _STRIPE, _STRIPE)],
                        dst_hbm.at[pl.ds(s * _STRIPE, _STRIPE)])

    @pl.when(s == _NS - 1)
    def _():
        pltpu.sync_copy(src_sh.at[pl.ds((_NS - 1) * _STRIPE, _LAST)],
                        dst_hbm.at[pl.ds((_NS - 1) * _STRIPE, _LAST)])


def _zero_acc_stripes(z_hbm, acc_sh, s):
    @pl.when(s < _NS - 1)
    def _():
        pltpu.sync_copy(z_hbm, acc_sh.at[pl.ds(s * _STRIPE, _STRIPE)])

    @pl.when(s == _NS - 1)
    def _():
        pltpu.sync_copy(z_hbm.at[pl.ds(0, _LAST)],
                        acc_sh.at[pl.ds((_NS - 1) * _STRIPE, _LAST)])


_DCPW = _NCHP // (_NC * _NS)


def _deg_body(s0, d0, s1, d1, zeros_n, ones_c, out,
              idx_v, ones_v, sh0, sh1, sh2, sh3, sem):
    c = lax.axis_index("c")
    s = lax.axis_index("s")
    wid = c * _NS + s

    @pl.when(s == 0)
    def _():
        pltpu.sync_copy(zeros_n, sh0)
        pltpu.sync_copy(zeros_n, sh1)
        pltpu.sync_copy(zeros_n, sh2)
        pltpu.sync_copy(zeros_n, sh3)

    pltpu.sync_copy(ones_c, ones_v)
    plsc.subcore_barrier()

    for ei, sh in ((s0, sh0), (d0, sh1), (s1, sh2), (d1, sh3)):
        pltpu.sync_copy(ei.at[pl.ds(wid * _DCPW, _DCPW)], idx_v)

        def grp(g, _, sh=sh):
            descs = []
            for b in range(10):
                descs.append(pltpu.async_copy(
                    ones_v, sh.at[idx_v.at[g * 10 + b]], sem, add=True))
            for d in descs:
                d.wait()
            return 0

        lax.fori_loop(0, _DCPW // 10, grp, 0)

    plsc.subcore_barrier()
    for a, sh in enumerate((sh0, sh1, sh2, sh3)):
        @pl.when(s < _NS - 1)
        def _(sh=sh, a=a):
            pltpu.sync_copy(sh.at[pl.ds(s * _DSTRIPE, _DSTRIPE)],
                            out.at[c, a, pl.ds(s * _DSTRIPE, _DSTRIPE)])

        @pl.when(s == _NS - 1)
        def _(sh=sh, a=a):
            pltpu.sync_copy(sh.at[pl.ds((_NS - 1) * _DSTRIPE, _DLAST)],
                            out.at[c, a, pl.ds((_NS - 1) * _DSTRIPE, _DLAST)])


@functools.lru_cache(maxsize=None)
def _deg_call():
    return pl.kernel(
        _deg_body,
        out_type=jax.ShapeDtypeStruct((_NC, 4, _NP), _f32),
        mesh=_sc_mesh(),
        scratch_types=[
            pltpu.VMEM((_DCPW, _CH), jnp.int32),
            pltpu.VMEM((_CH,), _f32),
            pltpu.VMEM_SHARED((_NP,), _f32),
            pltpu.VMEM_SHARED((_NP,), _f32),
            pltpu.VMEM_SHARED((_NP,), _f32),
            pltpu.VMEM_SHARED((_NP,), _f32),
            pltpu.SemaphoreType.DMA,
        ],
    )


def _scale_body(degp_ref, so_ref, si_ref):
    deg = degp_ref[0] + degp_ref[1]
    sc = lax.rsqrt(jnp.maximum(deg, 1.0))
    so_ref[...] = jnp.stack([sc[0, :_N], sc[2, :_N]], axis=1)
    si_ref[...] = jnp.stack([sc[1, :_N], sc[3, :_N]], axis=1)


_scale_call = pl.pallas_call(
    _scale_body,
    out_shape=(jax.ShapeDtypeStruct((_N, 2), _f32),
               jax.ShapeDtypeStruct((_N, 2), _f32)),
)


def _mm0_body(x_ref, so_ref, w0_ref, w1_ref, p0a, p0b, p1a, p1b):
    xb = x_ref[...]
    for r, (w_ref, pa, pb) in enumerate(((w0_ref, p0a, p0b),
                                         (w1_ref, p1a, p1b))):
        h = xb * so_ref[:, r:r + 1]
        p = jnp.dot(h, w_ref[...], preferred_element_type=_f32)
        pa[...] = p[:, :_DH]
        pb[...] = p[:, _DH:]


_mm0_call = pl.pallas_call(
    _mm0_body,
    grid=(_NB,),
    in_specs=[
        pl.BlockSpec((_ROWB, _D), lambda j: (j, 0)),
        pl.BlockSpec((_ROWB, 2), lambda j: (j, 0)),
        pl.BlockSpec((_D, _D), lambda j: (0, 0)),
        pl.BlockSpec((_D, _D), lambda j: (0, 0)),
    ],
    out_specs=[pl.BlockSpec((_ROWB, _DH), lambda j: (j, 0))] * 4,
    out_shape=[jax.ShapeDtypeStruct((_NROWS, _DH), _f32)] * 4,
)


def _mm_mid_body(split, a0a, a0b, a1a, a1b, si_ref, b_ref, so_ref,
                 w0_ref, w1_ref, *outs):
    si0 = si_ref[:, 0:1]
    si1 = si_ref[:, 1:2]
    ha = si0 * a0a[...] + si1 * a1a[...] + b_ref[0, :_DH]
    hb = si0 * a0b[...] + si1 * a1b[...] + b_ref[0, _DH:]
    h = jnp.concatenate([ha, hb], axis=1)
    for r, w_ref in enumerate((w0_ref, w1_ref)):
        p = jnp.dot(h * so_ref[:, r:r + 1], w_ref[...],
                    preferred_element_type=_f32)
        if split:
            outs[2 * r][...] = p[:, :_DH]
            outs[2 * r + 1][...] = p[:, _DH:]
        else:
            outs[r][...] = p


def _make_mm_mid(split, dout):
    n_out = 4 if split else 2
    ob = dout // 2 if split else dout
    return pl.pallas_call(
        functools.partial(_mm_mid_body, split),
        grid=(_NB,),
        in_specs=[
            pl.BlockSpec((_ROWB, _DH), lambda j: (j, 0)),
            pl.BlockSpec((_ROWB, _DH), lambda j: (j, 0)),
            pl.BlockSpec((_ROWB, _DH), lambda j: (j, 0)),
            pl.BlockSpec((_ROWB, _DH), lambda j: (j, 0)),
            pl.BlockSpec((_ROWB, 2), lambda j: (j, 0)),
            pl.BlockSpec((1, _D), lambda j: (0, 0)),
            pl.BlockSpec((_ROWB, 2), lambda j: (j, 0)),
            pl.BlockSpec((_D, dout), lambda j: (0, 0)),
            pl.BlockSpec((_D, dout), lambda j: (0, 0)),
        ],
        out_specs=[pl.BlockSpec((_ROWB, ob), lambda j: (j, 0))] * n_out,
        out_shape=[jax.ShapeDtypeStruct((_NROWS, ob), _f32)] * n_out,
    )


_mm_mid_call = _make_mm_mid(True, _D)
_mm_last_call = _make_mm_mid(False, _DH)


def _fin_body(p0A, p0B, p1A, p1B, si_ref, b_ref, out_ref):
    si0 = si_ref[:, 0:1]
    si1 = si_ref[:, 1:2]
    out_ref[...] = (si0 * (p0A[...] + p0B[...])
                    + si1 * (p1A[...] + p1B[...]) + b_ref[0, :])


_fin_call = pl.pallas_call(
    _fin_body,
    grid=(_NB,),
    in_specs=[
        pl.BlockSpec((_ROWB, _DH), lambda j: (j, 0)),
        pl.BlockSpec((_ROWB, _DH), lambda j: (j, 0)),
        pl.BlockSpec((_ROWB, _DH), lambda j: (j, 0)),
        pl.BlockSpec((_ROWB, _DH), lambda j: (j, 0)),
        pl.BlockSpec((_ROWB, 2), lambda j: (j, 0)),
        pl.BlockSpec((1, _DH), lambda j: (0, 0)),
    ],
    out_specs=pl.BlockSpec((_ROWB, _DH), lambda j: (j, 0)),
    out_shape=jax.ShapeDtypeStruct((_N, _DH), _f32),
)


def _prop_pipeline(tab, acc, src_v, dst_v, cpw, bufs, gsems, ssems):
    b0, b1 = bufs
    g0, g1 = gsems
    s0, s1 = ssems

    def wait_g(buf, gs, c):
        pltpu.make_async_copy(tab.at[src_v.at[c]], buf, gs).wait()

    def start_s(buf, ss, c):
        pltpu.async_copy(buf, acc.at[dst_v.at[c]], ss, add=True)

    def wait_s(buf, ss, c):
        pltpu.make_async_copy(buf, acc.at[dst_v.at[c]], ss).wait()

    def step(k, _):
        @pl.when(k > 0)
        def _():
            wait_s(b0, s0, 2 * k - 2)

        pltpu.async_copy(tab.at[src_v.at[2 * k]], b0, g0)

        @pl.when(k > 0)
        def _():
            wait_g(b1, g1, 2 * k - 1)
            start_s(b1, s1, 2 * k - 1)
            wait_s(b1, s1, 2 * k - 1)

        pltpu.async_copy(tab.at[src_v.at[2 * k + 1]], b1, g1)
        wait_g(b0, g0, 2 * k)
        start_s(b0, s0, 2 * k)
        return 0

    nsteps = cpw // 2
    lax.fori_loop(0, nsteps, step, 0)
    wait_g(b1, g1, cpw - 1)
    start_s(b1, s1, cpw - 1)
    wait_s(b0, s0, cpw - 2)
    wait_s(b1, s1, cpw - 1)


_CPW256 = _NCHP // _NS


def _prop256_body(ta, tb, src, dst, z2, out_a, out_b,
                  idx_s, idx_d, b0, b1,
                  acc, g0, g1, s0, s1):
    c = lax.axis_index("c")
    s = lax.axis_index("s")

    _zero_acc_stripes(z2, acc, s)
    plsc.subcore_barrier()

    bufs = (b0, b1)
    gsems = (g0, g1)
    ssems = (s0, s1)

    for phase in range(_CPW256 // _PCH):
        base = s * _CPW256 + phase * _PCH
        pltpu.sync_copy(src.at[pl.ds(base, _PCH)], idx_s)
        pltpu.sync_copy(dst.at[pl.ds(base, _PCH)], idx_d)

        @pl.when(c == 0)
        def _():
            _prop_pipeline(ta, acc, idx_s, idx_d, _PCH, bufs, gsems, ssems)

        @pl.when(c == 1)
        def _():
            _prop_pipeline(tb, acc, idx_s, idx_d, _PCH, bufs, gsems, ssems)

    plsc.subcore_barrier()

    @pl.when(c == 0)
    def _():
        _copy_stripes(acc, out_a, s)

    @pl.when(c == 1)
    def _():
        _copy_stripes(acc, out_b, s)


def _prop_scratch():
    return ([
        pltpu.VMEM((_PCH, _CH), jnp.int32),
        pltpu.VMEM((_PCH, _CH), jnp.int32),
    ] + [pltpu.VMEM((_CH, _DH), _f32)] * _NBUF
      + [pltpu.VMEM_SHARED((_NROWS, _DH), _f32)]
      + [pltpu.SemaphoreType.DMA] * (2 * _NBUF))


@functools.lru_cache(maxsize=None)
def _prop256_call():
    return pl.kernel(
        _prop256_body,
        out_type=(jax.ShapeDtypeStruct((_N, _DH), _f32),
                  jax.ShapeDtypeStruct((_N, _DH), _f32)),
        mesh=_sc_mesh(),
        scratch_types=_prop_scratch(),
    )


_CPW128 = _NCHP // (_NC * _NS)


def _prop128_body(tab, src, dst, z2, out_a, out_b,
                  idx_s, idx_d, b0, b1,
                  acc, g0, g1, s0, s1):
    c = lax.axis_index("c")
    s = lax.axis_index("s")
    wid = c * _NS + s

    _zero_acc_stripes(z2, acc, s)
    pltpu.sync_copy(src.at[pl.ds(wid * _CPW128, _CPW128)], idx_s)
    pltpu.sync_copy(dst.at[pl.ds(wid * _CPW128, _CPW128)], idx_d)
    plsc.subcore_barrier()

    _prop_pipeline(tab, acc, idx_s, idx_d, _CPW128,
                   (b0, b1), (g0, g1), (s0, s1))

    plsc.subcore_barrier()

    @pl.when(c == 0)
    def _():
        _copy_stripes(acc, out_a, s)

    @pl.when(c == 1)
    def _():
        _copy_stripes(acc, out_b, s)


@functools.lru_cache(maxsize=None)
def _prop128_call():
    return pl.kernel(
        _prop128_body,
        out_type=(jax.ShapeDtypeStruct((_N, _DH), _f32),
                  jax.ShapeDtypeStruct((_N, _DH), _f32)),
        mesh=_sc_mesh(),
        scratch_types=_prop_scratch(),
    )


def kernel(x, edge_index_rel0, edge_index_rel1,
           W0_r0, b0_r0, W0_r1, b0_r1,
           W1_r0, b1_r0, W1_r1, b1_r1,
           W2_r0, b2_r0, W2_r1, b2_r1):
    pad = _N + (jnp.arange(_EP - _E, dtype=jnp.int32) % (_NROWS - _N))

    def _padr(a):
        return jnp.concatenate([a, pad]).reshape(_NCHP, _CH)

    s0 = _padr(edge_index_rel0[0])
    d0 = _padr(edge_index_rel0[1])
    s1 = _padr(edge_index_rel1[0])
    d1 = _padr(edge_index_rel1[1])

    zeros_n = jnp.zeros((_NP,), _f32)
    ones_c = jnp.ones((_CH,), _f32)
    z2 = jnp.zeros((_STRIPE, _DH), _f32)

    degp = _deg_call()(s0, d0, s1, d1, zeros_n, ones_c)
    so, si = _scale_call(degp)

    b0 = (b0_r0 + b0_r1).reshape(1, _D)
    b1 = (b1_r0 + b1_r1).reshape(1, _D)
    b2 = (b2_r0 + b2_r1).reshape(1, _DH)

    prop256 = _prop256_call()
    prop128 = _prop128_call()

    p0a, p0b, p1a, p1b = _mm0_call(x, so, W0_r0, W0_r1)
    a0a, a0b = prop256(p0a, p0b, s0, d0, z2)
    a1a, a1b = prop256(p1a, p1b, s1, d1, z2)

    q0a, q0b, q1a, q1b = _mm_mid_call(a0a, a0b, a1a, a1b, si, b0, so,
                                      W1_r0, W1_r1)
    a0a, a0b = prop256(q0a, q0b, s0, d0, z2)
    a1a, a1b = prop256(q1a, q1b, s1, d1, z2)

    p0, p1 = _mm_last_call(a0a, a0b, a1a, a1b, si, b1, so, W2_r0, W2_r1)
    r0A, r0B = prop128(p0, s0, d0, z2)
    r1A, r1B = prop128(p1, s1, d1, z2)

    return _fin_call(r0A, r0B, r1A, r1B, si, b2)
